# baseline (speedup 1.0000x reference)
import functools
import jax
import jax.numpy as jnp
from jax import lax
from jax.experimental import pallas as pl
from jax.experimental.pallas import tpu as pltpu
from jax.experimental.pallas import tpu_sc as plsc


N_SAMPLES = 1024
N_CLUSTERS = 1024
D_EMB = 32
L = 16
NW = 32
SPW = N_SAMPLES // NW
NKB = N_CLUSTERS // L
ROW_W = N_CLUSTERS * D_EMB


def _proj_kernel(ctx_ref, q_ref, w1_ref, w2_ref, b_ref, p_ref):
    p_ref[...] = (
        jnp.dot(ctx_ref[...], w1_ref[...], preferred_element_type=jnp.float32)
        + jnp.dot(q_ref[...], w2_ref[...], preferred_element_type=jnp.float32)
        + b_ref[...]
    )


def _logsoftmax_kernel(x_ref, o_ref):
    x = x_ref[...]
    m = jnp.max(x, axis=1, keepdims=True)
    e = jnp.exp(x - m)
    s = jnp.sum(e, axis=1, keepdims=True)
    o_ref[...] = (x - m) - jnp.log(s)


NBUF = 4
QW = ROW_W // 4
QKB = 16


def _sc_logits_body(
    cc_hbm, proj_hbm, out_hbm,
    b0, b1, b2, b3,
    pbuf, tmp, stage, lrow,
    s0, s1, s2, s3,
    osem,
):
    bufs = (b0, b1, b2, b3)
    sems = (s0, s1, s2, s3)
    sid = lax.axis_index("s")
    wid = sid * 2 + lax.axis_index("c")
    base = wid * SPW

    pltpu.sync_copy(proj_hbm.at[pl.ds(base * D_EMB, SPW * D_EMB)], pbuf)

    for r in range(NBUF):
        pltpu.async_copy(cc_hbm.at[base, pl.ds(r * QW, QW)], bufs[r], sems[r])

    def body(g, _):
        n = base + g
        p0 = pbuf[pl.ds(g * D_EMB, L)]
        p1 = pbuf[pl.ds(g * D_EMB + L, L)]
        for ph in range(NBUF):
            buf = bufs[ph]

            pltpu.make_async_copy(
                cc_hbm.at[n, pl.ds(ph * QW, QW)], buf, sems[ph]
            ).wait()

            if ph == 0:

                @pl.when(g > 0)
                def _():
                    pltpu.make_async_copy(lrow, out_hbm.at[n - 1], osem).wait()

            @plsc.parallel_loop(0, QKB)
            def kb_body(kb):
                o = kb * (L * D_EMB)
                t0 = (kb % 8) * 2048
                sreg = (kb % 8) * 32
                for j in range(L):
                    tj = t0 + j * 128
                    a = buf[pl.ds(o + j * D_EMB, L)] * p0 + buf[
                        pl.ds(o + j * D_EMB + L, L)
                    ] * p1
                    tmp[pl.ds(tj, L)] = a
                    bb = a + tmp[pl.ds(tj + 8, L)]
                    tmp[pl.ds(tj + 32, L)] = bb
                    c = bb + tmp[pl.ds(tj + 32 + 4, L)]
                    tmp[pl.ds(tj + 64, L)] = c
                    d = c + tmp[pl.ds(tj + 64 + 2, L)]
                    tmp[pl.ds(tj + 96, L)] = d
                    e = d + tmp[pl.ds(tj + 96 + 1, L)]
                    stage[pl.ds(sreg + j, L)] = e
                w = stage[pl.ds(sreg, L)]
                lrow[pl.ds(ph * (QKB * L) + kb * L, L)] = w

            if ph == NBUF - 1:
                pltpu.async_copy(lrow, out_hbm.at[n], osem)

            @pl.when(g < SPW - 1)
            def _():
                pltpu.async_copy(
                    cc_hbm.at[n + 1, pl.ds(ph * QW, QW)], buf, sems[ph]
                )

        return 0

    lax.fori_loop(0, SPW, body, 0)
    pltpu.make_async_copy(lrow, out_hbm.at[base + SPW - 1], osem).wait()


_sc_logits = functools.partial(
    pl.kernel,
    out_type=jax.ShapeDtypeStruct((N_SAMPLES, N_CLUSTERS), jnp.float32),
    mesh=plsc.VectorSubcoreMesh(core_axis_name="c", subcore_axis_name="s"),
    scratch_types=[pltpu.VMEM((QW,), jnp.float32) for _ in range(NBUF)]
    + [
        pltpu.VMEM((SPW * D_EMB,), jnp.float32),
        pltpu.VMEM((8 * 2048,), jnp.float32),
        pltpu.VMEM((8 * 32 + L,), jnp.float32),
        pltpu.VMEM((N_CLUSTERS,), jnp.float32),
    ]
    + [pltpu.SemaphoreType.DMA for _ in range(NBUF)]
    + [pltpu.SemaphoreType.DMA],
)(_sc_logits_body)


@jax.jit
def kernel(context, query, cluster_centers, W, b):
    n, dc = context.shape
    w1 = W[:dc]
    w2 = W[dc:]
    b_row = b.reshape(1, -1)

    proj = pl.pallas_call(
        _proj_kernel,
        out_shape=jax.ShapeDtypeStruct((n, D_EMB), jnp.float32),
    )(context, query, w1, w2, b_row)

    cc_r = cluster_centers.reshape(n, ROW_W)
    logits = _sc_logits(cc_r, proj.reshape(-1))

    blk_r = 128
    log_probs = pl.pallas_call(
        _logsoftmax_kernel,
        grid=(n // blk_r,),
        in_specs=[pl.BlockSpec((blk_r, N_CLUSTERS), lambda i: (i, 0))],
        out_specs=pl.BlockSpec((blk_r, N_CLUSTERS), lambda i: (i, 0)),
        out_shape=jax.ShapeDtypeStruct((n, N_CLUSTERS), jnp.float32),
    )(logits)

    return (logits, log_probs)


# speedup vs baseline: 1.2830x; 1.2830x over previous
import functools
import jax
import jax.numpy as jnp
from jax import lax
from jax.experimental import pallas as pl
from jax.experimental.pallas import tpu as pltpu
from jax.experimental.pallas import tpu_sc as plsc


N_SAMPLES = 1024
N_CLUSTERS = 1024
D_EMB = 32
L = 16
NW = 32
SPW = N_SAMPLES // NW
NKB = N_CLUSTERS // L
ROW_W = N_CLUSTERS * D_EMB


def _proj_kernel(ctx_ref, q_ref, w1_ref, w2_ref, b_ref, p_ref):
    p_ref[...] = (
        jnp.dot(ctx_ref[...], w1_ref[...], preferred_element_type=jnp.float32)
        + jnp.dot(q_ref[...], w2_ref[...], preferred_element_type=jnp.float32)
        + b_ref[...]
    )


def _logsoftmax_kernel(x_ref, o_ref):
    x = x_ref[...]
    m = jnp.max(x, axis=1, keepdims=True)
    e = jnp.exp(x - m)
    s = jnp.sum(e, axis=1, keepdims=True)
    o_ref[...] = (x - m) - jnp.log(s)


NBUF = 4
HW = ROW_W // 2
HKB = 32


def _sc_logits_body(
    cc_hbm, proj_hbm, out_hbm,
    b0, b1, b2, b3,
    pbuf, tmp, stage, lrow0, lrow1,
    s0, s1, s2, s3,
    osem0, osem1,
):
    bufs = (b0, b1, b2, b3)
    sems = (s0, s1, s2, s3)
    sid = lax.axis_index("s")
    wid = sid * 2 + lax.axis_index("c")
    base = wid * SPW

    pltpu.sync_copy(proj_hbm.at[pl.ds(base * D_EMB, SPW * D_EMB)], pbuf)

    for r in range(NBUF):
        pltpu.async_copy(
            cc_hbm.at[base + r // 2, pl.ds((r % 2) * HW, HW)], bufs[r], sems[r]
        )

    def body(g, _):
        for ph in range(NBUF):
            hpart = ph % 2
            i_local = 2 * g + ph // 2
            n = base + i_local
            buf = bufs[ph]
            lrow = lrow0 if ph < 2 else lrow1
            osem = osem0 if ph < 2 else osem1

            pltpu.make_async_copy(
                cc_hbm.at[n, pl.ds(hpart * HW, HW)], buf, sems[ph]
            ).wait()

            if hpart == 0:

                @pl.when(g > 0)
                def _():
                    pltpu.make_async_copy(lrow, out_hbm.at[n - 2], osem).wait()

            p0 = pbuf[pl.ds(i_local * D_EMB, L)]
            p1 = pbuf[pl.ds(i_local * D_EMB + L, L)]

            @plsc.parallel_loop(0, HKB)
            def kb_body(kb):
                o = kb * (L * D_EMB)
                t0 = (kb % 8) * 2048
                sreg = (kb % 8) * 32
                for j in range(L):
                    tj = t0 + j * 128
                    a = buf[pl.ds(o + j * D_EMB, L)] * p0 + buf[
                        pl.ds(o + j * D_EMB + L, L)
                    ] * p1
                    tmp[pl.ds(tj, L)] = a
                    bb = a + tmp[pl.ds(tj + 8, L)]
                    tmp[pl.ds(tj + 32, L)] = bb
                    c = bb + tmp[pl.ds(tj + 32 + 4, L)]
                    tmp[pl.ds(tj + 64, L)] = c
                    d = c + tmp[pl.ds(tj + 64 + 2, L)]
                    tmp[pl.ds(tj + 96, L)] = d
                    e = d + tmp[pl.ds(tj + 96 + 1, L)]
                    stage[pl.ds(sreg + j, L)] = e
                w = stage[pl.ds(sreg, L)]
                lrow[pl.ds(hpart * (HKB * L) + kb * L, L)] = w

            if hpart == 1:
                pltpu.async_copy(lrow, out_hbm.at[n], osem)

            @pl.when(g < SPW // 2 - 1)
            def _():
                pltpu.async_copy(
                    cc_hbm.at[n + 2, pl.ds(hpart * HW, HW)], buf, sems[ph]
                )

        return 0

    lax.fori_loop(0, SPW // 2, body, 0)
    pltpu.make_async_copy(lrow0, out_hbm.at[base + SPW - 2], osem0).wait()
    pltpu.make_async_copy(lrow1, out_hbm.at[base + SPW - 1], osem1).wait()


_sc_logits = functools.partial(
    pl.kernel,
    out_type=jax.ShapeDtypeStruct((N_SAMPLES, N_CLUSTERS), jnp.float32),
    mesh=plsc.VectorSubcoreMesh(core_axis_name="c", subcore_axis_name="s"),
    scratch_types=[pltpu.VMEM((HW,), jnp.float32) for _ in range(NBUF)]
    + [
        pltpu.VMEM((SPW * D_EMB,), jnp.float32),
        pltpu.VMEM((8 * 2048,), jnp.float32),
        pltpu.VMEM((8 * 32 + L,), jnp.float32),
        pltpu.VMEM((N_CLUSTERS,), jnp.float32),
        pltpu.VMEM((N_CLUSTERS,), jnp.float32),
    ]
    + [pltpu.SemaphoreType.DMA for _ in range(NBUF)]
    + [pltpu.SemaphoreType.DMA, pltpu.SemaphoreType.DMA],
)(_sc_logits_body)


@jax.jit
def kernel(context, query, cluster_centers, W, b):
    n, dc = context.shape
    w1 = W[:dc]
    w2 = W[dc:]
    b_row = b.reshape(1, -1)

    proj = pl.pallas_call(
        _proj_kernel,
        out_shape=jax.ShapeDtypeStruct((n, D_EMB), jnp.float32),
    )(context, query, w1, w2, b_row)

    cc_r = cluster_centers.reshape(n, ROW_W)
    logits = _sc_logits(cc_r, proj.reshape(-1))

    blk_r = 128
    log_probs = pl.pallas_call(
        _logsoftmax_kernel,
        grid=(n // blk_r,),
        in_specs=[pl.BlockSpec((blk_r, N_CLUSTERS), lambda i: (i, 0))],
        out_specs=pl.BlockSpec((blk_r, N_CLUSTERS), lambda i: (i, 0)),
        out_shape=jax.ShapeDtypeStruct((n, N_CLUSTERS), jnp.float32),
    )(logits)

    return (logits, log_probs)
